# baseline (speedup 1.0000x reference)
import functools

import jax
import jax.numpy as jnp
from jax import lax
from jax.experimental import pallas as pl
from jax.experimental.pallas import tpu as pltpu
from jax.experimental.pallas import tpu_sc as plsc

_PREFIX = 2048
_NBUF = 2
_CROWS = 16


def _make_sc_copy(b, s, d):
    info = plsc.get_sparse_core_info()
    nc, ns = info.num_cores, info.num_subcores
    nw = nc * ns
    rows_per_w = _PREFIX * b // nw
    w_per_b = nw // b
    nch = rows_per_w // _CROWS
    lookahead = _NBUF - 1

    mesh = plsc.VectorSubcoreMesh(core_axis_name="c", subcore_axis_name="s")

    @functools.partial(
        pl.kernel,
        mesh=mesh,
        out_type=jax.ShapeDtypeStruct((b, _PREFIX, d), jnp.float32),
        scratch_types=(
            [pltpu.VMEM((_CROWS, d), jnp.float32)] * _NBUF
            + [pltpu.SemaphoreType.DMA] * (2 * _NBUF)
        ),
    )
    def sc_copy(x_hbm, out_hbm, *scratch):
        bufs = scratch[:_NBUF]
        in_sems = scratch[_NBUF : 2 * _NBUF]
        out_sems = scratch[2 * _NBUF :]
        wid = lax.axis_index("s") * nc + lax.axis_index("c")
        bi = wid // w_per_b
        r0 = (wid % w_per_b) * rows_per_w

        def start_gather(i):
            slot = i % _NBUF
            return pltpu.async_copy(
                x_hbm.at[bi, pl.ds(r0 + i * _CROWS, _CROWS), :],
                bufs[slot],
                in_sems[slot],
            )

        def start_scatter(i):
            slot = i % _NBUF
            return pltpu.async_copy(
                bufs[slot],
                out_hbm.at[bi, pl.ds(r0 + i * _CROWS, _CROWS), :],
                out_sems[slot],
            )

        hout = [None] * _NBUF
        hin = [None] * _NBUF
        for j in range(min(lookahead, nch)):
            hin[j % _NBUF] = start_gather(j)
        for i in range(nch):
            slot = i % _NBUF
            hin[slot].wait()
            hout[slot] = start_scatter(i)
            j = i + lookahead
            if j < nch:
                sj = j % _NBUF
                if hout[sj] is not None:
                    hout[sj].wait()
                    hout[sj] = None
                hin[sj] = start_gather(j)
        for slot in range(_NBUF):
            if hout[slot] is not None:
                hout[slot].wait()

    return sc_copy


def kernel(input_tensor, dim, prefix_len):
    b, s, d = input_tensor.shape
    return _make_sc_copy(b, s, d)(input_tensor)


# speedup vs baseline: 1.0274x; 1.0274x over previous
import functools

import jax
import jax.numpy as jnp
from jax import lax
from jax.experimental import pallas as pl
from jax.experimental.pallas import tpu as pltpu
from jax.experimental.pallas import tpu_sc as plsc

_PREFIX = 2048
_NBUF = 3
_CROWS = 16


def _make_sc_copy(b, s, d):
    info = plsc.get_sparse_core_info()
    nc, ns = info.num_cores, info.num_subcores
    nw = nc * ns
    rows_per_w = _PREFIX * b // nw
    w_per_b = nw // b
    nch = rows_per_w // _CROWS
    lookahead = _NBUF - 1

    mesh = plsc.VectorSubcoreMesh(core_axis_name="c", subcore_axis_name="s")

    @functools.partial(
        pl.kernel,
        mesh=mesh,
        out_type=jax.ShapeDtypeStruct((b, _PREFIX, d), jnp.float32),
        scratch_types=(
            [pltpu.VMEM((_CROWS, d), jnp.float32)] * _NBUF
            + [pltpu.SemaphoreType.DMA] * (2 * _NBUF)
        ),
    )
    def sc_copy(x_hbm, out_hbm, *scratch):
        bufs = scratch[:_NBUF]
        in_sems = scratch[_NBUF : 2 * _NBUF]
        out_sems = scratch[2 * _NBUF :]
        wid = lax.axis_index("s") * nc + lax.axis_index("c")
        bi = wid // w_per_b
        r0 = (wid % w_per_b) * rows_per_w

        def start_gather(i):
            slot = i % _NBUF
            return pltpu.async_copy(
                x_hbm.at[bi, pl.ds(r0 + i * _CROWS, _CROWS), :],
                bufs[slot],
                in_sems[slot],
            )

        def start_scatter(i):
            slot = i % _NBUF
            return pltpu.async_copy(
                bufs[slot],
                out_hbm.at[bi, pl.ds(r0 + i * _CROWS, _CROWS), :],
                out_sems[slot],
            )

        hout = [None] * _NBUF
        hin = [None] * _NBUF
        for j in range(min(lookahead, nch)):
            hin[j % _NBUF] = start_gather(j)
        for i in range(nch):
            slot = i % _NBUF
            hin[slot].wait()
            hout[slot] = start_scatter(i)
            j = i + lookahead
            if j < nch:
                sj = j % _NBUF
                if hout[sj] is not None:
                    hout[sj].wait()
                    hout[sj] = None
                hin[sj] = start_gather(j)
        for slot in range(_NBUF):
            if hout[slot] is not None:
                hout[slot].wait()

    return sc_copy


def kernel(input_tensor, dim, prefix_len):
    b, s, d = input_tensor.shape
    return _make_sc_copy(b, s, d)(input_tensor)


# speedup vs baseline: 1.0624x; 1.0341x over previous
import functools

import jax
import jax.numpy as jnp
from jax import lax
from jax.experimental import pallas as pl
from jax.experimental.pallas import tpu as pltpu
from jax.experimental.pallas import tpu_sc as plsc

_PREFIX = 2048
_NBUF = 2
_CROWS = 16


def _make_sc_copy(b, s, d):
    info = plsc.get_sparse_core_info()
    nc, ns = info.num_cores, info.num_subcores
    nw = nc * ns
    rows_per_w = _PREFIX * b // nw
    w_per_b = nw // b
    nch = rows_per_w // _CROWS
    nbuf = 2 * _NBUF
    lookahead = nbuf - 1

    mesh = plsc.VectorSubcoreMesh(core_axis_name="c", subcore_axis_name="s")

    @functools.partial(
        pl.kernel,
        mesh=mesh,
        out_type=jax.ShapeDtypeStruct((b, _PREFIX, d), jnp.float32),
        scratch_types=(
            [pltpu.VMEM((_CROWS, d), jnp.float32)] * _NBUF
            + [pltpu.VMEM_SHARED((ns, _CROWS, d), jnp.float32)] * _NBUF
            + [pltpu.SemaphoreType.DMA] * (4 * _NBUF)
        ),
    )
    def sc_copy(x_hbm, out_hbm, *scratch):
        sid = lax.axis_index("s")
        bufs = []
        for k in range(_NBUF):
            bufs.append(scratch[k])
            bufs.append(scratch[_NBUF + k].at[sid])
        in_sems = scratch[2 * _NBUF : 2 * _NBUF + nbuf]
        out_sems = scratch[2 * _NBUF + nbuf :]
        wid = sid * nc + lax.axis_index("c")
        bi = wid // w_per_b
        r0 = (wid % w_per_b) * rows_per_w

        def start_gather(i):
            slot = i % nbuf
            return pltpu.async_copy(
                x_hbm.at[bi, pl.ds(r0 + i * _CROWS, _CROWS), :],
                bufs[slot],
                in_sems[slot],
            )

        def start_scatter(i):
            slot = i % nbuf
            return pltpu.async_copy(
                bufs[slot],
                out_hbm.at[bi, pl.ds(r0 + i * _CROWS, _CROWS), :],
                out_sems[slot],
            )

        hout = [None] * nbuf
        hin = [None] * nbuf
        for j in range(min(lookahead, nch)):
            hin[j % nbuf] = start_gather(j)
        for i in range(nch):
            slot = i % nbuf
            hin[slot].wait()
            hout[slot] = start_scatter(i)
            j = i + lookahead
            if j < nch:
                sj = j % nbuf
                if hout[sj] is not None:
                    hout[sj].wait()
                    hout[sj] = None
                hin[sj] = start_gather(j)
        for slot in range(nbuf):
            if hout[slot] is not None:
                hout[slot].wait()

    return sc_copy


def kernel(input_tensor, dim, prefix_len):
    b, s, d = input_tensor.shape
    return _make_sc_copy(b, s, d)(input_tensor)


# speedup vs baseline: 1.0702x; 1.0073x over previous
import functools

import jax
import jax.numpy as jnp
from jax import lax
from jax.experimental import pallas as pl
from jax.experimental.pallas import tpu as pltpu
from jax.experimental.pallas import tpu_sc as plsc

_PREFIX = 2048
_NTILE = 1
_NSP = 2
_CROWS = 16


def _make_sc_copy(b, s, d):
    info = plsc.get_sparse_core_info()
    nc, ns = info.num_cores, info.num_subcores
    nw = nc * ns
    rows_per_w = _PREFIX * b // nw
    w_per_b = nw // b
    nch = rows_per_w // _CROWS
    nbuf = _NTILE + _NSP
    lookahead = nbuf - 1

    mesh = plsc.VectorSubcoreMesh(core_axis_name="c", subcore_axis_name="s")

    @functools.partial(
        pl.kernel,
        mesh=mesh,
        out_type=jax.ShapeDtypeStruct((b, _PREFIX, d), jnp.float32),
        scratch_types=(
            [pltpu.VMEM((_CROWS, d), jnp.float32)] * _NTILE
            + [pltpu.VMEM_SHARED((ns, _CROWS, d), jnp.float32)] * _NSP
            + [pltpu.SemaphoreType.DMA] * (2 * (_NTILE + _NSP))
        ),
    )
    def sc_copy(x_hbm, out_hbm, *scratch):
        sid = lax.axis_index("s")
        bufs = [scratch[k] for k in range(_NTILE)]
        bufs += [scratch[_NTILE + k].at[sid] for k in range(_NSP)]
        in_sems = scratch[nbuf : 2 * nbuf]
        out_sems = scratch[2 * nbuf :]
        wid = sid * nc + lax.axis_index("c")
        bi = wid // w_per_b
        r0 = (wid % w_per_b) * rows_per_w

        def start_gather(i):
            slot = i % nbuf
            return pltpu.async_copy(
                x_hbm.at[bi, pl.ds(r0 + i * _CROWS, _CROWS), :],
                bufs[slot],
                in_sems[slot],
            )

        def start_scatter(i):
            slot = i % nbuf
            return pltpu.async_copy(
                bufs[slot],
                out_hbm.at[bi, pl.ds(r0 + i * _CROWS, _CROWS), :],
                out_sems[slot],
            )

        hout = [None] * nbuf
        hin = [None] * nbuf
        for j in range(min(lookahead, nch)):
            hin[j % nbuf] = start_gather(j)
        for i in range(nch):
            slot = i % nbuf
            hin[slot].wait()
            hout[slot] = start_scatter(i)
            j = i + lookahead
            if j < nch:
                sj = j % nbuf
                if hout[sj] is not None:
                    hout[sj].wait()
                    hout[sj] = None
                hin[sj] = start_gather(j)
        for slot in range(nbuf):
            if hout[slot] is not None:
                hout[slot].wait()

    return sc_copy


def kernel(input_tensor, dim, prefix_len):
    b, s, d = input_tensor.shape
    return _make_sc_copy(b, s, d)(input_tensor)
